# baseline (speedup 1.0000x reference)
import functools
import random

import jax
import jax.numpy as jnp
import numpy as np
from jax.experimental import pallas as pl
from jax.experimental.pallas import tpu as pltpu

_NUM_DOCS = 5
_NEG_INF = -1e30


def _doc_lengths(seq_len: int, num_docs: int = _NUM_DOCS):
    rng = random.Random(0)
    lengths = [1] * num_docs
    for _ in range(seq_len - num_docs):
        lengths[rng.randint(0, num_docs - 1)] += 1
    return lengths


@functools.lru_cache(maxsize=None)
def _schedule(seq_len: int, bq: int, bk: int):
    bounds = np.concatenate(
        [[0], np.cumsum(_doc_lengths(seq_len))]).astype(np.int32)
    nq = seq_len // bq
    qidx, kidx = [], []
    for qb in range(nq):
        lo, hi = qb * bq, (qb + 1) * bq - 1
        d0 = int(np.searchsorted(bounds, lo, "right")) - 1
        d1 = int(np.searchsorted(bounds, hi, "right")) - 1
        ks = int(bounds[d0]) // bk
        ke = -(-int(bounds[d1 + 1]) // bk)
        for kb in range(ks, ke):
            qidx.append(qb)
            kidx.append(kb)
    return (tuple(int(b) for b in bounds),
            np.asarray(qidx, np.int32), np.asarray(kidx, np.int32))


def _flash_body(qi_ref, ki_ref, q_ref, k_ref, v_ref, o_ref,
                m_ref, l_ref, acc_ref, *, bounds, bq, bk, num_pairs, scale):
    p = pl.program_id(1)
    qb = qi_ref[p]
    kb = ki_ref[p]

    prev_q = qi_ref[jnp.maximum(p - 1, 0)]
    next_q = qi_ref[jnp.minimum(p + 1, num_pairs - 1)]
    is_first = jnp.logical_or(p == 0, prev_q != qb)
    is_last = jnp.logical_or(p == num_pairs - 1, next_q != qb)

    q = q_ref[0, 0]
    k = k_ref[0, 0]
    v = v_ref[0, 0]

    s = jax.lax.dot_general(
        q, k, (((1,), (1,)), ((), ())),
        preferred_element_type=jnp.float32) * scale

    row = qb * bq + jax.lax.broadcasted_iota(jnp.int32, (bq, 1), 0)
    col = kb * bk + jax.lax.broadcasted_iota(jnp.int32, (1, bk), 1)
    start = jnp.zeros((bq, 1), jnp.int32)
    end = jnp.zeros((bq, 1), jnp.int32)
    for j in range(len(bounds) - 1):
        in_doc = jnp.logical_and(row >= bounds[j], row < bounds[j + 1])
        start = jnp.where(in_doc, bounds[j], start)
        end = jnp.where(in_doc, bounds[j + 1], end)
    mask = jnp.logical_and(col >= start, col < end)
    s = jnp.where(mask, s, _NEG_INF)

    m_prev = jnp.where(is_first, _NEG_INF, m_ref[:, :1])
    l_prev = jnp.where(is_first, 0.0, l_ref[:, :1])
    acc_prev = jnp.where(is_first, 0.0, acc_ref[...])

    m_cur = jnp.max(s, axis=1, keepdims=True)
    m_new = jnp.maximum(m_prev, m_cur)
    alpha = jnp.exp(m_prev - m_new)
    pmat = jnp.exp(s - m_new)
    l_new = l_prev * alpha + jnp.sum(pmat, axis=1, keepdims=True)
    acc_new = acc_prev * alpha + jax.lax.dot_general(
        pmat, v, (((1,), (0,)), ((), ())),
        preferred_element_type=jnp.float32)

    m_ref[...] = jnp.broadcast_to(m_new, m_ref.shape)
    l_ref[...] = jnp.broadcast_to(l_new, l_ref.shape)
    acc_ref[...] = acc_new

    @pl.when(is_last)
    def _():
        o_ref[0, 0] = acc_new / l_new


def kernel(q, k, v):
    b, h, s, d = q.shape
    assert b == 1
    bq, bk = 256, 256
    bounds, qidx, kidx = _schedule(s, bq, bk)
    num_pairs = len(qidx)
    scale = 1.0 / float(np.sqrt(d))

    grid = (h, num_pairs)

    def q_map(hh, p, qi, ki):
        return (0, hh, qi[p], 0)

    def kv_map(hh, p, qi, ki):
        return (0, hh, ki[p], 0)

    body = functools.partial(
        _flash_body, bounds=bounds, bq=bq, bk=bk,
        num_pairs=num_pairs, scale=scale)

    out = pl.pallas_call(
        body,
        grid_spec=pltpu.PrefetchScalarGridSpec(
            num_scalar_prefetch=2,
            grid=grid,
            in_specs=[
                pl.BlockSpec((1, 1, bq, d), q_map),
                pl.BlockSpec((1, 1, bk, d), kv_map),
                pl.BlockSpec((1, 1, bk, d), kv_map),
            ],
            out_specs=pl.BlockSpec((1, 1, bq, d), q_map),
            scratch_shapes=[
                pltpu.VMEM((bq, 128), jnp.float32),
                pltpu.VMEM((bq, 128), jnp.float32),
                pltpu.VMEM((bq, d), jnp.float32),
            ],
        ),
        out_shape=jax.ShapeDtypeStruct((b, h, s, d), jnp.float32),
        compiler_params=pltpu.CompilerParams(
            dimension_semantics=("parallel", "arbitrary")),
    )(jnp.asarray(qidx), jnp.asarray(kidx), q, k, v)
    return out
